# Initial kernel scaffold; baseline (speedup 1.0000x reference)
import jax
import jax.numpy as jnp
from jax.experimental import pallas as pl


def kernel(x, enc_w1, enc_b1, enc_w2, enc_b2, enc_w3, enc_b3, emb, dec_w1, dec_b1, dec_w2, dec_b2, dec_w3, dec_b3, dec_w4, dec_b4):
    raise NotImplementedError("write your pallas kernel here")


# speedup vs baseline: 1.0304x; 1.0304x over previous
import functools

import jax
import jax.numpy as jnp
from jax import lax
from jax.experimental import pallas as pl
from jax.experimental.pallas import tpu as pltpu


def _conv2d(x, w, b, stride, pad):
    out = lax.conv_general_dilated(
        x, w, (stride, stride), [(pad, pad), (pad, pad)],
        dimension_numbers=('NCHW', 'OIHW', 'NCHW'))
    return out + b[None, :, None, None]


def _conv_transpose2d(x, w, b, stride, pad, k):
    w2 = jnp.flip(w, (2, 3)).transpose(1, 0, 2, 3)
    p = k - 1 - pad
    out = lax.conv_general_dilated(
        x, w2, (1, 1), [(p, p), (p, p)], lhs_dilation=(stride, stride),
        dimension_numbers=('NCHW', 'OIHW', 'NCHW'))
    return out + b[None, :, None, None]


def _vq_kernel(z_ref, emb_ref, zq_ref, codes_ref, ssq_ref):
    z = z_ref[...]
    emb = emb_ref[...]
    score = jnp.dot(z, emb.T, preferred_element_type=jnp.float32) * (-2.0)
    score = score + jnp.sum(emb * emb, axis=1)[None, :]
    codes = jnp.argmin(score, axis=1).astype(jnp.int32)
    onehot = (lax.broadcasted_iota(jnp.int32, score.shape, 1)
              == codes[:, None]).astype(jnp.float32)
    zq = jnp.dot(onehot, emb, preferred_element_type=jnp.float32)
    zq_ref[...] = zq
    codes_ref[...] = codes[:, None]
    diff = zq - z
    part = jnp.sum(diff * diff, keepdims=True)

    @pl.when(pl.program_id(0) == 0)
    def _init():
        ssq_ref[...] = jnp.zeros_like(ssq_ref)

    ssq_ref[...] += part


def _vq(z_flat, emb):
    m = z_flat.shape[0]
    bm = 1344
    grid = m // bm
    zq, codes, ssq = pl.pallas_call(
        _vq_kernel,
        grid=(grid,),
        in_specs=[
            pl.BlockSpec((bm, 128), lambda i: (i, 0)),
            pl.BlockSpec((512, 128), lambda i: (0, 0)),
        ],
        out_specs=[
            pl.BlockSpec((bm, 128), lambda i: (i, 0)),
            pl.BlockSpec((bm, 1), lambda i: (i, 0)),
            pl.BlockSpec((1, 1), lambda i: (0, 0)),
        ],
        out_shape=[
            jax.ShapeDtypeStruct((m, 128), jnp.float32),
            jax.ShapeDtypeStruct((m, 1), jnp.int32),
            jax.ShapeDtypeStruct((1, 1), jnp.float32),
        ],
    )(z_flat, emb)
    return zq, codes[:, 0], ssq[0, 0]


def kernel(x, enc_w1, enc_b1, enc_w2, enc_b2, enc_w3, enc_b3, emb,
           dec_w1, dec_b1, dec_w2, dec_b2, dec_w3, dec_b3, dec_w4, dec_b4):
    beta = 0.25
    h = jax.nn.relu(_conv2d(x, enc_w1, enc_b1, 2, 1))
    h = jax.nn.relu(_conv2d(h, enc_w2, enc_b2, 2, 1))
    z_e = jax.nn.relu(_conv2d(h, enc_w3, enc_b3, 2, 1))
    B, C, H, W = z_e.shape
    z_flat = z_e.transpose(0, 2, 3, 1).reshape(-1, C)
    zq_flat, codes, ssq = _vq(z_flat, emb)
    vq_loss = (1.0 + beta) * ssq / (B * C * H * W)
    z_q = zq_flat.reshape(B, H, W, C).transpose(0, 3, 1, 2)
    d = jax.nn.relu(_conv_transpose2d(z_q, dec_w1, dec_b1, 2, 1, 4))
    d = jax.nn.relu(_conv_transpose2d(d, dec_w2, dec_b2, 2, 1, 4))
    d = jax.nn.relu(_conv_transpose2d(d, dec_w3, dec_b3, 2, 1, 4))
    x_hat = jax.nn.sigmoid(_conv2d(d, dec_w4, dec_b4, 1, 1))
    x_hat = x_hat[:, :, :, :172]
    return (x_hat, vq_loss, codes.reshape(B, H, W))
